# Initial kernel scaffold; baseline (speedup 1.0000x reference)
import jax
import jax.numpy as jnp
from jax.experimental import pallas as pl


def kernel(input):
    raise NotImplementedError("write your pallas kernel here")


# speedup vs baseline: 28.4030x; 28.4030x over previous
import functools

import numpy as np
import jax
import jax.numpy as jnp
from jax.experimental import pallas as pl

_K = 20
_N_COEFFS = 231
_GRID_H = 128
_OUT_H = 64
_K_PAD = 512


def _build_weights() -> np.ndarray:
    k1s, k2s = [], []
    for k1 in range(_K + 1):
        for k2 in range(_K + 1 - k1):
            k1s.append(k1)
            k2s.append(k2)
    k1s = np.asarray(k1s)
    k2s = np.asarray(k2s)
    n = np.arange(_OUT_H)
    theta = (2.0 * np.pi / _GRID_H) * (
        k1s[:, None, None] * n[None, :, None]
        + k2s[:, None, None] * n[None, None, :]
    )
    scale = np.where(k2s == 0, 1.0, 2.0) / np.sqrt(float(_N_COEFFS))
    w_real = (scale[:, None, None] * np.cos(theta)).reshape(_N_COEFFS, -1)
    w_imag = (-scale[:, None, None] * np.sin(theta)).reshape(_N_COEFFS, -1)
    w = np.concatenate(
        [w_real, w_imag,
         np.zeros((_K_PAD - 2 * _N_COEFFS, _OUT_H * _OUT_H))], axis=0)
    return np.ascontiguousarray(w, dtype=np.float32)


_W = _build_weights()


def _matmul_kernel(x_ref, w_ref, o_ref):
    o_ref[...] = jnp.dot(x_ref[...], w_ref[...],
                         preferred_element_type=jnp.float32)


@functools.partial(jax.jit, static_argnums=(1,))
def _apply(x_pad, m):
    n_total = _OUT_H * _OUT_H
    n_tile = 1024
    grid = (n_total // n_tile,)
    w = jnp.asarray(_W)
    return pl.pallas_call(
        _matmul_kernel,
        grid=grid,
        in_specs=[
            pl.BlockSpec((m, _K_PAD), lambda j: (0, 0)),
            pl.BlockSpec((_K_PAD, n_tile), lambda j: (0, j)),
        ],
        out_specs=pl.BlockSpec((m, n_tile), lambda j: (0, j)),
        out_shape=jax.ShapeDtypeStruct((m, n_total), jnp.float32),
    )(x_pad, w)


def kernel(input):
    b = input.shape[0]
    c = int(np.prod(input.shape[1:])) // (2 * _N_COEFFS)
    m = b * c
    x = input.reshape(m, 2 * _N_COEFFS)
    x_pad = jnp.pad(x, ((0, 0), (0, _K_PAD - 2 * _N_COEFFS)))
    y = _apply(x_pad, m)
    return y.reshape(b, c, _OUT_H, _OUT_H)


# speedup vs baseline: 29.5184x; 1.0393x over previous
import functools

import numpy as np
import jax
import jax.numpy as jnp
from jax.experimental import pallas as pl

_K = 20
_N_COEFFS = 231
_GRID_H = 128
_OUT_H = 64
_K_PAD = 512


def _build_weights() -> np.ndarray:
    k1s, k2s = [], []
    for k1 in range(_K + 1):
        for k2 in range(_K + 1 - k1):
            k1s.append(k1)
            k2s.append(k2)
    k1s = np.asarray(k1s)
    k2s = np.asarray(k2s)
    n = np.arange(_OUT_H)
    theta = (2.0 * np.pi / _GRID_H) * (
        k1s[:, None, None] * n[None, :, None]
        + k2s[:, None, None] * n[None, None, :]
    )
    scale = np.where(k2s == 0, 1.0, 2.0) / np.sqrt(float(_N_COEFFS))
    w_real = (scale[:, None, None] * np.cos(theta)).reshape(_N_COEFFS, -1)
    w_imag = (-scale[:, None, None] * np.sin(theta)).reshape(_N_COEFFS, -1)
    w = np.concatenate(
        [w_real, w_imag,
         np.zeros((_K_PAD - 2 * _N_COEFFS, _OUT_H * _OUT_H))], axis=0)
    return np.ascontiguousarray(w, dtype=np.float32)


_W = _build_weights()
_W_BF16 = jnp.asarray(_W, dtype=jnp.bfloat16)


def _matmul_kernel(x_ref, w_ref, o_ref):
    o_ref[...] = jnp.dot(x_ref[...], w_ref[...],
                         preferred_element_type=jnp.float32)


@functools.partial(jax.jit, static_argnums=(1,))
def _apply(x_pad, m):
    n_total = _OUT_H * _OUT_H
    n_tile = 1024
    grid = (n_total // n_tile,)
    w = _W_BF16
    return pl.pallas_call(
        _matmul_kernel,
        grid=grid,
        in_specs=[
            pl.BlockSpec((m, _K_PAD), lambda j: (0, 0)),
            pl.BlockSpec((_K_PAD, n_tile), lambda j: (0, j)),
        ],
        out_specs=pl.BlockSpec((m, n_tile), lambda j: (0, j)),
        out_shape=jax.ShapeDtypeStruct((m, n_total), jnp.float32),
    )(x_pad, w)


def kernel(input):
    b = input.shape[0]
    c = int(np.prod(input.shape[1:])) // (2 * _N_COEFFS)
    m = b * c
    x = input.reshape(m, 2 * _N_COEFFS)
    x_pad = jnp.pad(x, ((0, 0), (0, _K_PAD - 2 * _N_COEFFS)))
    x_pad = x_pad.astype(jnp.bfloat16)
    y = _apply(x_pad, m)
    return y.reshape(b, c, _OUT_H, _OUT_H)


# speedup vs baseline: 31.5461x; 1.0687x over previous
import functools

import numpy as np
import jax
import jax.numpy as jnp
from jax.experimental import pallas as pl

_K = 20
_N_COEFFS = 231
_GRID_H = 128
_OUT_H = 64
_K_PAD = 512


def _build_weights() -> np.ndarray:
    k1s, k2s = [], []
    for k1 in range(_K + 1):
        for k2 in range(_K + 1 - k1):
            k1s.append(k1)
            k2s.append(k2)
    k1s = np.asarray(k1s)
    k2s = np.asarray(k2s)
    n = np.arange(_OUT_H)
    theta = (2.0 * np.pi / _GRID_H) * (
        k1s[:, None, None] * n[None, :, None]
        + k2s[:, None, None] * n[None, None, :]
    )
    scale = np.where(k2s == 0, 1.0, 2.0) / np.sqrt(float(_N_COEFFS))
    w_real = (scale[:, None, None] * np.cos(theta)).reshape(_N_COEFFS, -1)
    w_imag = (-scale[:, None, None] * np.sin(theta)).reshape(_N_COEFFS, -1)
    w = np.concatenate(
        [w_real, w_imag,
         np.zeros((_K_PAD - 2 * _N_COEFFS, _OUT_H * _OUT_H))], axis=0)
    return np.ascontiguousarray(w, dtype=np.float32)


_W = _build_weights()
_W_BF16 = _W.astype(jnp.bfloat16)


def _matmul_kernel(x_ref, w_ref, o_ref):
    x = x_ref[...].astype(jnp.bfloat16)
    o_ref[...] = jnp.dot(x, w_ref[...],
                         preferred_element_type=jnp.float32)


@functools.partial(jax.jit, static_argnums=(1,))
def _apply(x, m):
    n_total = _OUT_H * _OUT_H
    n_tile = 1024
    grid = (n_total // n_tile,)
    k = 2 * _N_COEFFS
    w = jnp.asarray(_W_BF16[:k])
    return pl.pallas_call(
        _matmul_kernel,
        grid=grid,
        in_specs=[
            pl.BlockSpec((m, k), lambda j: (0, 0)),
            pl.BlockSpec((k, n_tile), lambda j: (0, j)),
        ],
        out_specs=pl.BlockSpec((m, n_tile), lambda j: (0, j)),
        out_shape=jax.ShapeDtypeStruct((m, n_total), jnp.float32),
    )(x, w)


def kernel(input):
    b = input.shape[0]
    c = int(np.prod(input.shape[1:])) // (2 * _N_COEFFS)
    m = b * c
    x = input.reshape(m, 2 * _N_COEFFS)
    y = _apply(x, m)
    return y.reshape(b, c, _OUT_H, _OUT_H)
